# Initial kernel scaffold; baseline (speedup 1.0000x reference)
import jax
import jax.numpy as jnp
from jax.experimental import pallas as pl


def kernel(edge_index, edge_type, node_emb, rel_w1, root_w1, bias1, rel_w2, root_w2, bias2):
    raise NotImplementedError("write your pallas kernel here")


# speedup vs baseline: 12.0991x; 12.0991x over previous
import functools

import jax
import jax.numpy as jnp
from jax import lax
from jax.experimental import pallas as pl
from jax.experimental.pallas import tpu as pltpu
from jax.experimental.pallas import tpu_sc as plsc

N = 10000
R = 8
D = 128
E = 320000
RN = R * N

NC = 2
NS = 16
L = 16

EDGES_PER_CORE = E // NC
EDGES_PER_TILE = EDGES_PER_CORE // NS
W = 80
NWIN = EDGES_PER_TILE // W

_MESH = plsc.VectorSubcoreMesh(core_axis_name="c", subcore_axis_name="s")
_SC_PARAMS = pltpu.CompilerParams(use_tc_tiling_on_sc=False)


CNT_ROWS_PER_TILE = RN // NS
CNT_ZROWS = 1000


@functools.partial(
    pl.kernel,
    out_type=jax.ShapeDtypeStruct((NC, RN, L), jnp.float32),
    mesh=_MESH,
    scratch_types=[
        pltpu.VMEM((W,), jnp.int32),
        pltpu.VMEM((W,), jnp.int32),
        pltpu.VMEM((W,), jnp.int32),
        pltpu.VMEM((W, L), jnp.float32),
        pltpu.VMEM((CNT_ZROWS, L), jnp.float32),
        pltpu.VMEM_SHARED((RN, L), jnp.float32),
    ],
    compiler_params=_SC_PARAMS,
)
def _sc_count(dst_hbm, rel_hbm, out_hbm, dst_v, rel_v, idx_v, ones_v, zbuf,
              acc_sh):
    core = lax.axis_index("c")
    sid = lax.axis_index("s")

    @pl.loop(0, W)
    def _(k):
        ones_v[k, :] = jnp.ones((L,), jnp.float32)

    @pl.loop(0, CNT_ZROWS)
    def _(i):
        zbuf[i, :] = jnp.zeros((L,), jnp.float32)

    rowstart = sid * CNT_ROWS_PER_TILE
    for j in range(CNT_ROWS_PER_TILE // CNT_ZROWS):
        pltpu.sync_copy(zbuf, acc_sh.at[pl.ds(rowstart + j * CNT_ZROWS,
                                              CNT_ZROWS)])
    plsc.subcore_barrier()

    base = core * EDGES_PER_CORE + sid * EDGES_PER_TILE

    @pl.loop(0, NWIN)
    def _(w):
        off = base + w * W
        pltpu.sync_copy(dst_hbm.at[pl.ds(off, W)], dst_v)
        pltpu.sync_copy(rel_hbm.at[pl.ds(off, W)], rel_v)

        @pl.loop(0, W, step=L)
        def _(j):
            sl = pl.ds(j, L)
            idx_v[sl] = rel_v[sl] * N + dst_v[sl]

        pltpu.sync_copy(ones_v, acc_sh.at[idx_v], add=True)

    plsc.subcore_barrier()
    for j in range(CNT_ROWS_PER_TILE // CNT_ZROWS):
        sl = pl.ds(rowstart + j * CNT_ZROWS, CNT_ZROWS)
        pltpu.sync_copy(acc_sh.at[sl], out_hbm.at[core, sl])


AGG_ROWS_PER_TILE = N // NS
AGG_ZROWS = 125


@functools.partial(
    pl.kernel,
    out_type=jax.ShapeDtypeStruct((NC, N, D), jnp.float32),
    mesh=_MESH,
    scratch_types=[
        pltpu.VMEM((W,), jnp.int32),
        pltpu.VMEM((W,), jnp.int32),
        pltpu.VMEM((W,), jnp.int32),
        pltpu.VMEM((W,), jnp.int32),
        pltpu.VMEM((W,), jnp.int32),
        pltpu.VMEM((W, D), jnp.float32),
        pltpu.VMEM((W, L), jnp.float32),
        pltpu.VMEM((AGG_ZROWS, D), jnp.float32),
        pltpu.VMEM_SHARED((N, D), jnp.float32),
        pltpu.SemaphoreType.DMA,
        pltpu.SemaphoreType.DMA,
    ],
    compiler_params=_SC_PARAMS,
)
def _sc_agg(y_hbm, inv_hbm, src_hbm, dst_hbm, rel_hbm, out_hbm,
            src_v, dst_v, rel_v, gidx_v, widx_v, rows_v, w_v, zbuf, acc_sh,
            sem_a, sem_b):
    core = lax.axis_index("c")
    sid = lax.axis_index("s")

    @pl.loop(0, AGG_ZROWS)
    def _(i):
        @pl.loop(0, D, step=L)
        def _(j):
            zbuf[i, pl.ds(j, L)] = jnp.zeros((L,), jnp.float32)

    rowstart = sid * AGG_ROWS_PER_TILE
    for j in range(AGG_ROWS_PER_TILE // AGG_ZROWS):
        pltpu.sync_copy(zbuf, acc_sh.at[pl.ds(rowstart + j * AGG_ZROWS,
                                              AGG_ZROWS)])
    plsc.subcore_barrier()

    base = core * EDGES_PER_CORE + sid * EDGES_PER_TILE

    @pl.loop(0, NWIN)
    def _(w):
        off = base + w * W
        pltpu.sync_copy(src_hbm.at[pl.ds(off, W)], src_v)
        pltpu.sync_copy(dst_hbm.at[pl.ds(off, W)], dst_v)
        pltpu.sync_copy(rel_hbm.at[pl.ds(off, W)], rel_v)

        @pl.loop(0, W, step=L)
        def _(j):
            sl = pl.ds(j, L)
            rel16 = rel_v[sl]
            gidx_v[sl] = rel16 * N + src_v[sl]
            widx_v[sl] = rel16 * N + dst_v[sl]

        cp_rows = pltpu.async_copy(y_hbm.at[gidx_v], rows_v, sem_a)
        cp_w = pltpu.async_copy(inv_hbm.at[widx_v], w_v, sem_b)
        cp_rows.wait()
        cp_w.wait()

        @pl.loop(0, W)
        def _(k):
            wk = w_v[k, :]
            for j in range(D // L):
                sl = pl.ds(j * L, L)
                rows_v[k, sl] = rows_v[k, sl] * wk

        pltpu.sync_copy(rows_v, acc_sh.at[dst_v], add=True)

    plsc.subcore_barrier()
    for j in range(AGG_ROWS_PER_TILE // AGG_ZROWS):
        sl = pl.ds(rowstart + j * AGG_ZROWS, AGG_ZROWS)
        pltpu.sync_copy(acc_sh.at[sl], out_hbm.at[core, sl])


NB = 5
BN = N // NB


def _transform_body(x_ref, w_ref, y_ref):
    y_ref[...] = jnp.dot(x_ref[...], w_ref[0],
                         preferred_element_type=jnp.float32)


def _tc_transform(x, rel_w):
    return pl.pallas_call(
        _transform_body,
        grid=(R, NB),
        in_specs=[
            pl.BlockSpec((BN, D), lambda r, b: (b, 0)),
            pl.BlockSpec((1, D, D), lambda r, b: (r, 0, 0)),
        ],
        out_specs=pl.BlockSpec((BN, D), lambda r, b: (r * NB + b, 0)),
        out_shape=jax.ShapeDtypeStruct((RN, D), jnp.float32),
    )(x, rel_w)


def _prep_body(c0_ref, c1_ref, o_ref):
    o_ref[...] = 1.0 / jnp.maximum(c0_ref[...] + c1_ref[...], 1.0)


def _tc_prep(cnt_part):
    c = cnt_part.reshape(NC, RN * L // D, D)
    inv = pl.pallas_call(
        _prep_body,
        grid=(5,),
        in_specs=[
            pl.BlockSpec((RN * L // D // 5, D), lambda b: (b, 0)),
            pl.BlockSpec((RN * L // D // 5, D), lambda b: (b, 0)),
        ],
        out_specs=pl.BlockSpec((RN * L // D // 5, D), lambda b: (b, 0)),
        out_shape=jax.ShapeDtypeStruct((RN * L // D, D), jnp.float32),
    )(c[0], c[1])
    return inv.reshape(RN, L)


def _combine_body(x_ref, rw_ref, b_ref, p0_ref, p1_ref, o_ref, *, act):
    v = jnp.dot(x_ref[...], rw_ref[...], preferred_element_type=jnp.float32)
    v = v + b_ref[...] + p0_ref[...] + p1_ref[...]
    o_ref[...] = jnp.maximum(v, 0.0) if act else v


def _tc_combine(x, root_w, bias, part, act):
    return pl.pallas_call(
        functools.partial(_combine_body, act=act),
        grid=(NB,),
        in_specs=[
            pl.BlockSpec((BN, D), lambda b: (b, 0)),
            pl.BlockSpec((D, D), lambda b: (0, 0)),
            pl.BlockSpec((1, D), lambda b: (0, 0)),
            pl.BlockSpec((BN, D), lambda b: (b, 0)),
            pl.BlockSpec((BN, D), lambda b: (b, 0)),
        ],
        out_specs=pl.BlockSpec((BN, D), lambda b: (b, 0)),
        out_shape=jax.ShapeDtypeStruct((N, D), jnp.float32),
    )(x, root_w, bias.reshape(1, D), part[0], part[1])


def kernel(edge_index, edge_type, node_emb, rel_w1, root_w1, bias1,
           rel_w2, root_w2, bias2):
    src = edge_index[0]
    dst = edge_index[1]
    rel = edge_type

    cnt_part = _sc_count(dst, rel)
    inv = _tc_prep(cnt_part)

    y1 = _tc_transform(node_emb, rel_w1)
    p1 = _sc_agg(y1, inv, src, dst, rel)
    x2 = _tc_combine(node_emb, root_w1, bias1, p1, act=True)

    y2 = _tc_transform(x2, rel_w2)
    p2 = _sc_agg(y2, inv, src, dst, rel)
    out = _tc_combine(x2, root_w2, bias2, p2, act=False)
    return out


# speedup vs baseline: 15.4946x; 1.2806x over previous
import functools

import jax
import jax.numpy as jnp
from jax import lax
from jax.experimental import pallas as pl
from jax.experimental.pallas import tpu as pltpu
from jax.experimental.pallas import tpu_sc as plsc

N = 10000
R = 8
D = 128
E = 320000
RN = R * N

NC = 2
NS = 16
L = 16

EDGES_PER_CORE = E // NC
EDGES_PER_TILE = EDGES_PER_CORE // NS
W = 80
NWIN = EDGES_PER_TILE // W

_MESH = plsc.VectorSubcoreMesh(core_axis_name="c", subcore_axis_name="s")
_SC_PARAMS = pltpu.CompilerParams(use_tc_tiling_on_sc=False)


CNT_ROWS_PER_TILE = RN // NS
CNT_ZROWS = 1000


@functools.partial(
    pl.kernel,
    out_type=jax.ShapeDtypeStruct((NC, RN, L), jnp.float32),
    mesh=_MESH,
    scratch_types=[
        pltpu.VMEM((W,), jnp.int32),
        pltpu.VMEM((W,), jnp.int32),
        pltpu.VMEM((W,), jnp.int32),
        pltpu.VMEM((W, L), jnp.float32),
        pltpu.VMEM((CNT_ZROWS, L), jnp.float32),
        pltpu.VMEM_SHARED((RN, L), jnp.float32),
    ],
    compiler_params=_SC_PARAMS,
)
def _sc_count(dst_hbm, rel_hbm, out_hbm, dst_v, rel_v, idx_v, ones_v, zbuf,
              acc_sh):
    core = lax.axis_index("c")
    sid = lax.axis_index("s")

    @pl.loop(0, W)
    def _(k):
        ones_v[k, :] = jnp.ones((L,), jnp.float32)

    @pl.loop(0, CNT_ZROWS)
    def _(i):
        zbuf[i, :] = jnp.zeros((L,), jnp.float32)

    rowstart = sid * CNT_ROWS_PER_TILE
    for j in range(CNT_ROWS_PER_TILE // CNT_ZROWS):
        pltpu.sync_copy(zbuf, acc_sh.at[pl.ds(rowstart + j * CNT_ZROWS,
                                              CNT_ZROWS)])
    plsc.subcore_barrier()

    base = core * EDGES_PER_CORE + sid * EDGES_PER_TILE

    @pl.loop(0, NWIN)
    def _(w):
        off = base + w * W
        pltpu.sync_copy(dst_hbm.at[pl.ds(off, W)], dst_v)
        pltpu.sync_copy(rel_hbm.at[pl.ds(off, W)], rel_v)

        @pl.loop(0, W, step=L)
        def _(j):
            sl = pl.ds(j, L)
            idx_v[sl] = rel_v[sl] * N + dst_v[sl]

        pltpu.sync_copy(ones_v, acc_sh.at[idx_v], add=True)

    plsc.subcore_barrier()
    for j in range(CNT_ROWS_PER_TILE // CNT_ZROWS):
        sl = pl.ds(rowstart + j * CNT_ZROWS, CNT_ZROWS)
        pltpu.sync_copy(acc_sh.at[sl], out_hbm.at[core, sl])


AGG_ROWS_PER_TILE = N // NS
AGG_ZROWS = 125


def _agg_buf_types():
    return [
        pltpu.VMEM((W,), jnp.int32),
        pltpu.VMEM((W,), jnp.int32),
        pltpu.VMEM((W,), jnp.int32),
        pltpu.VMEM((W,), jnp.int32),
        pltpu.VMEM((W,), jnp.int32),
        pltpu.VMEM((W, D), jnp.float32),
        pltpu.VMEM((W, L), jnp.float32),
        pltpu.SemaphoreType.DMA,
        pltpu.SemaphoreType.DMA,
    ]


@functools.partial(
    pl.kernel,
    out_type=jax.ShapeDtypeStruct((NC, N, D), jnp.float32),
    mesh=_MESH,
    scratch_types=_agg_buf_types() + _agg_buf_types() + [
        pltpu.VMEM((AGG_ZROWS, D), jnp.float32),
        pltpu.VMEM_SHARED((N, D), jnp.float32),
    ],
    compiler_params=_SC_PARAMS,
)
def _sc_agg(y_hbm, inv_hbm, src_hbm, dst_hbm, rel_hbm, out_hbm,
            *bufs_and_more):
    buf_a = bufs_and_more[0:9]
    buf_b = bufs_and_more[9:18]
    zbuf, acc_sh = bufs_and_more[18], bufs_and_more[19]
    core = lax.axis_index("c")
    sid = lax.axis_index("s")

    @pl.loop(0, AGG_ZROWS)
    def _(i):
        @pl.loop(0, D, step=L)
        def _(j):
            zbuf[i, pl.ds(j, L)] = jnp.zeros((L,), jnp.float32)

    rowstart = sid * AGG_ROWS_PER_TILE
    for j in range(AGG_ROWS_PER_TILE // AGG_ZROWS):
        pltpu.sync_copy(zbuf, acc_sh.at[pl.ds(rowstart + j * AGG_ZROWS,
                                              AGG_ZROWS)])
    plsc.subcore_barrier()

    base = core * EDGES_PER_CORE + sid * EDGES_PER_TILE

    def start(w, buf):
        (src_v, dst_v, rel_v, gidx_v, widx_v, rows_v, w_v, sem_a,
         sem_b) = buf
        off = base + w * W
        pltpu.sync_copy(src_hbm.at[pl.ds(off, W)], src_v)
        pltpu.sync_copy(dst_hbm.at[pl.ds(off, W)], dst_v)
        pltpu.sync_copy(rel_hbm.at[pl.ds(off, W)], rel_v)

        @pl.loop(0, W, step=L)
        def _(j):
            sl = pl.ds(j, L)
            rel16 = rel_v[sl]
            gidx_v[sl] = rel16 * N + src_v[sl]
            widx_v[sl] = rel16 * N + dst_v[sl]

        pltpu.async_copy(y_hbm.at[gidx_v], rows_v, sem_a)
        pltpu.async_copy(inv_hbm.at[widx_v], w_v, sem_b)

    def finish(buf):
        (src_v, dst_v, rel_v, gidx_v, widx_v, rows_v, w_v, sem_a,
         sem_b) = buf
        pltpu.make_async_copy(y_hbm.at[gidx_v], rows_v, sem_a).wait()
        pltpu.make_async_copy(inv_hbm.at[widx_v], w_v, sem_b).wait()

        @pl.loop(0, W, unroll=4)
        def _(k):
            wk = w_v[k, :]
            for j in range(D // L):
                sl = pl.ds(j * L, L)
                rows_v[k, sl] = rows_v[k, sl] * wk

        pltpu.sync_copy(rows_v, acc_sh.at[dst_v], add=True)

    start(0, buf_a)

    @pl.loop(0, NWIN - 1, step=2)
    def _(w):
        start(w + 1, buf_b)
        finish(buf_a)
        start(w + 2, buf_a)
        finish(buf_b)

    finish(buf_a)

    plsc.subcore_barrier()
    for j in range(AGG_ROWS_PER_TILE // AGG_ZROWS):
        sl = pl.ds(rowstart + j * AGG_ZROWS, AGG_ZROWS)
        pltpu.sync_copy(acc_sh.at[sl], out_hbm.at[core, sl])


NB = 5
BN = N // NB


def _transform_body(x_ref, w_ref, y_ref):
    y_ref[...] = jnp.dot(x_ref[...], w_ref[0],
                         preferred_element_type=jnp.float32)


def _tc_transform(x, rel_w):
    return pl.pallas_call(
        _transform_body,
        grid=(R, NB),
        in_specs=[
            pl.BlockSpec((BN, D), lambda r, b: (b, 0)),
            pl.BlockSpec((1, D, D), lambda r, b: (r, 0, 0)),
        ],
        out_specs=pl.BlockSpec((BN, D), lambda r, b: (r * NB + b, 0)),
        out_shape=jax.ShapeDtypeStruct((RN, D), jnp.float32),
    )(x, rel_w)


def _prep_body(c0_ref, c1_ref, o_ref):
    o_ref[...] = 1.0 / jnp.maximum(c0_ref[...] + c1_ref[...], 1.0)


def _tc_prep(cnt_part):
    c = cnt_part.reshape(NC, RN * L // D, D)
    inv = pl.pallas_call(
        _prep_body,
        grid=(5,),
        in_specs=[
            pl.BlockSpec((RN * L // D // 5, D), lambda b: (b, 0)),
            pl.BlockSpec((RN * L // D // 5, D), lambda b: (b, 0)),
        ],
        out_specs=pl.BlockSpec((RN * L // D // 5, D), lambda b: (b, 0)),
        out_shape=jax.ShapeDtypeStruct((RN * L // D, D), jnp.float32),
    )(c[0], c[1])
    return inv.reshape(RN, L)


def _combine_body(x_ref, rw_ref, b_ref, p0_ref, p1_ref, o_ref, *, act):
    v = jnp.dot(x_ref[...], rw_ref[...], preferred_element_type=jnp.float32)
    v = v + b_ref[...] + p0_ref[...] + p1_ref[...]
    o_ref[...] = jnp.maximum(v, 0.0) if act else v


def _tc_combine(x, root_w, bias, part, act):
    return pl.pallas_call(
        functools.partial(_combine_body, act=act),
        grid=(NB,),
        in_specs=[
            pl.BlockSpec((BN, D), lambda b: (b, 0)),
            pl.BlockSpec((D, D), lambda b: (0, 0)),
            pl.BlockSpec((1, D), lambda b: (0, 0)),
            pl.BlockSpec((BN, D), lambda b: (b, 0)),
            pl.BlockSpec((BN, D), lambda b: (b, 0)),
        ],
        out_specs=pl.BlockSpec((BN, D), lambda b: (b, 0)),
        out_shape=jax.ShapeDtypeStruct((N, D), jnp.float32),
    )(x, root_w, bias.reshape(1, D), part[0], part[1])


def kernel(edge_index, edge_type, node_emb, rel_w1, root_w1, bias1,
           rel_w2, root_w2, bias2):
    src = edge_index[0]
    dst = edge_index[1]
    rel = edge_type

    cnt_part = _sc_count(dst, rel)
    inv = _tc_prep(cnt_part)

    y1 = _tc_transform(node_emb, rel_w1)
    p1 = _sc_agg(y1, inv, src, dst, rel)
    x2 = _tc_combine(node_emb, root_w1, bias1, p1, act=True)

    y2 = _tc_transform(x2, rel_w2)
    p2 = _sc_agg(y2, inv, src, dst, rel)
    out = _tc_combine(x2, root_w2, bias2, p2, act=False)
    return out


# speedup vs baseline: 16.9018x; 1.0908x over previous
import functools

import jax
import jax.numpy as jnp
from jax import lax
from jax.experimental import pallas as pl
from jax.experimental.pallas import tpu as pltpu
from jax.experimental.pallas import tpu_sc as plsc

N = 10000
R = 8
D = 128
E = 320000
RN = R * N

NC = 2
NS = 16
L = 16

EDGES_PER_CORE = E // NC
EDGES_PER_TILE = EDGES_PER_CORE // NS
W = 80
NWIN = EDGES_PER_TILE // W

_MESH = plsc.VectorSubcoreMesh(core_axis_name="c", subcore_axis_name="s")
_SC_PARAMS = pltpu.CompilerParams(use_tc_tiling_on_sc=False)


CNT_ROWS_PER_TILE = RN // NS
CNT_ZROWS = 1000


@functools.partial(
    pl.kernel,
    out_type=jax.ShapeDtypeStruct((NC, RN, L), jnp.float32),
    mesh=_MESH,
    scratch_types=[
        pltpu.VMEM((W,), jnp.int32),
        pltpu.VMEM((W,), jnp.int32),
        pltpu.VMEM((W,), jnp.int32),
        pltpu.SemaphoreType.DMA,
        pltpu.SemaphoreType.DMA,
        pltpu.VMEM((W,), jnp.int32),
        pltpu.VMEM((W,), jnp.int32),
        pltpu.VMEM((W,), jnp.int32),
        pltpu.SemaphoreType.DMA,
        pltpu.SemaphoreType.DMA,
        pltpu.VMEM((W, L), jnp.float32),
        pltpu.VMEM((CNT_ZROWS, L), jnp.float32),
        pltpu.VMEM_SHARED((RN, L), jnp.float32),
    ],
    compiler_params=_SC_PARAMS,
)
def _sc_count(dst_hbm, rel_hbm, out_hbm,
              dst_a, rel_a, idx_a, sa1, sa2,
              dst_b, rel_b, idx_b, sb1, sb2,
              ones_v, zbuf, acc_sh):
    buf_a = (dst_a, rel_a, idx_a, sa1, sa2)
    buf_b = (dst_b, rel_b, idx_b, sb1, sb2)
    core = lax.axis_index("c")
    sid = lax.axis_index("s")

    @pl.loop(0, W)
    def _(k):
        ones_v[k, :] = jnp.ones((L,), jnp.float32)

    @pl.loop(0, CNT_ZROWS)
    def _(i):
        zbuf[i, :] = jnp.zeros((L,), jnp.float32)

    rowstart = sid * CNT_ROWS_PER_TILE
    for j in range(CNT_ROWS_PER_TILE // CNT_ZROWS):
        pltpu.sync_copy(zbuf, acc_sh.at[pl.ds(rowstart + j * CNT_ZROWS,
                                              CNT_ZROWS)])
    plsc.subcore_barrier()

    base = core * EDGES_PER_CORE + sid * EDGES_PER_TILE

    def start(w, buf):
        dst_v, rel_v, idx_v, s1, s2 = buf
        off = base + w * W
        pltpu.async_copy(dst_hbm.at[pl.ds(off, W)], dst_v, s1)
        pltpu.async_copy(rel_hbm.at[pl.ds(off, W)], rel_v, s2)

    def finish(w, buf):
        dst_v, rel_v, idx_v, s1, s2 = buf
        off = base + w * W
        pltpu.make_async_copy(dst_hbm.at[pl.ds(off, W)], dst_v, s1).wait()
        pltpu.make_async_copy(rel_hbm.at[pl.ds(off, W)], rel_v, s2).wait()

        @pl.loop(0, W, step=L)
        def _(j):
            sl = pl.ds(j, L)
            idx_v[sl] = rel_v[sl] * N + dst_v[sl]

        pltpu.sync_copy(ones_v, acc_sh.at[idx_v], add=True)

    start(0, buf_a)

    @pl.loop(0, NWIN - 1, step=2)
    def _(w):
        start(w + 1, buf_b)
        finish(w, buf_a)
        start(w + 2, buf_a)
        finish(w + 1, buf_b)

    finish(NWIN - 1, buf_a)

    plsc.subcore_barrier()
    for j in range(CNT_ROWS_PER_TILE // CNT_ZROWS):
        sl = pl.ds(rowstart + j * CNT_ZROWS, CNT_ZROWS)
        pltpu.sync_copy(acc_sh.at[sl], out_hbm.at[core, sl])


AGG_ROWS_PER_TILE = N // NS
AGG_ZROWS = 125


def _agg_buf_types():
    return [
        pltpu.VMEM((W,), jnp.int32),
        pltpu.VMEM((W,), jnp.int32),
        pltpu.VMEM((W,), jnp.int32),
        pltpu.VMEM((W,), jnp.int32),
        pltpu.VMEM((W,), jnp.int32),
        pltpu.VMEM((W, D), jnp.float32),
        pltpu.VMEM((W, L), jnp.float32),
        pltpu.SemaphoreType.DMA,
        pltpu.SemaphoreType.DMA,
    ]


@functools.partial(
    pl.kernel,
    out_type=jax.ShapeDtypeStruct((NC, N, D), jnp.float32),
    mesh=_MESH,
    scratch_types=_agg_buf_types() + _agg_buf_types() + [
        pltpu.VMEM((AGG_ZROWS, D), jnp.float32),
        pltpu.VMEM_SHARED((N, D), jnp.float32),
    ],
    compiler_params=_SC_PARAMS,
)
def _sc_agg(y_hbm, inv_hbm, src_hbm, dst_hbm, rel_hbm, out_hbm,
            *bufs_and_more):
    buf_a = bufs_and_more[0:9]
    buf_b = bufs_and_more[9:18]
    zbuf, acc_sh = bufs_and_more[18], bufs_and_more[19]
    core = lax.axis_index("c")
    sid = lax.axis_index("s")

    @pl.loop(0, AGG_ZROWS)
    def _(i):
        @pl.loop(0, D, step=L)
        def _(j):
            zbuf[i, pl.ds(j, L)] = jnp.zeros((L,), jnp.float32)

    rowstart = sid * AGG_ROWS_PER_TILE
    for j in range(AGG_ROWS_PER_TILE // AGG_ZROWS):
        pltpu.sync_copy(zbuf, acc_sh.at[pl.ds(rowstart + j * AGG_ZROWS,
                                              AGG_ZROWS)])
    plsc.subcore_barrier()

    base = core * EDGES_PER_CORE + sid * EDGES_PER_TILE

    def start(w, buf):
        (src_v, dst_v, rel_v, gidx_v, widx_v, rows_v, w_v, sem_a,
         sem_b) = buf
        off = base + w * W
        pltpu.sync_copy(src_hbm.at[pl.ds(off, W)], src_v)
        pltpu.sync_copy(dst_hbm.at[pl.ds(off, W)], dst_v)
        pltpu.sync_copy(rel_hbm.at[pl.ds(off, W)], rel_v)

        @pl.loop(0, W, step=L)
        def _(j):
            sl = pl.ds(j, L)
            rel16 = rel_v[sl]
            gidx_v[sl] = rel16 * N + src_v[sl]
            widx_v[sl] = rel16 * N + dst_v[sl]

        pltpu.async_copy(y_hbm.at[gidx_v], rows_v, sem_a)
        pltpu.async_copy(inv_hbm.at[widx_v], w_v, sem_b)

    def finish(buf):
        (src_v, dst_v, rel_v, gidx_v, widx_v, rows_v, w_v, sem_a,
         sem_b) = buf
        pltpu.make_async_copy(y_hbm.at[gidx_v], rows_v, sem_a).wait()
        pltpu.make_async_copy(inv_hbm.at[widx_v], w_v, sem_b).wait()

        @pl.loop(0, W, unroll=4)
        def _(k):
            wk = w_v[k, :]
            for j in range(D // L):
                sl = pl.ds(j * L, L)
                rows_v[k, sl] = rows_v[k, sl] * wk

        pltpu.sync_copy(rows_v, acc_sh.at[dst_v], add=True)

    start(0, buf_a)

    @pl.loop(0, NWIN - 1, step=2)
    def _(w):
        start(w + 1, buf_b)
        finish(buf_a)
        start(w + 2, buf_a)
        finish(buf_b)

    finish(buf_a)

    plsc.subcore_barrier()
    for j in range(AGG_ROWS_PER_TILE // AGG_ZROWS):
        sl = pl.ds(rowstart + j * AGG_ZROWS, AGG_ZROWS)
        pltpu.sync_copy(acc_sh.at[sl], out_hbm.at[core, sl])


NB = 5
BN = N // NB


def _transform_body(x_ref, w_ref, y_ref):
    y_ref[...] = jnp.dot(x_ref[...], w_ref[0],
                         preferred_element_type=jnp.float32)


def _tc_transform(x, rel_w):
    return pl.pallas_call(
        _transform_body,
        grid=(R, NB),
        in_specs=[
            pl.BlockSpec((BN, D), lambda r, b: (b, 0)),
            pl.BlockSpec((1, D, D), lambda r, b: (r, 0, 0)),
        ],
        out_specs=pl.BlockSpec((BN, D), lambda r, b: (r * NB + b, 0)),
        out_shape=jax.ShapeDtypeStruct((RN, D), jnp.float32),
    )(x, rel_w)


def _prep_body(c0_ref, c1_ref, o_ref):
    o_ref[...] = 1.0 / jnp.maximum(c0_ref[...] + c1_ref[...], 1.0)


def _tc_prep(cnt_part):
    c = cnt_part.reshape(NC, RN * L // D, D)
    inv = pl.pallas_call(
        _prep_body,
        grid=(5,),
        in_specs=[
            pl.BlockSpec((RN * L // D // 5, D), lambda b: (b, 0)),
            pl.BlockSpec((RN * L // D // 5, D), lambda b: (b, 0)),
        ],
        out_specs=pl.BlockSpec((RN * L // D // 5, D), lambda b: (b, 0)),
        out_shape=jax.ShapeDtypeStruct((RN * L // D, D), jnp.float32),
    )(c[0], c[1])
    return inv.reshape(RN, L)


def _combine_body(x_ref, rw_ref, b_ref, p0_ref, p1_ref, o_ref, *, act):
    v = jnp.dot(x_ref[...], rw_ref[...], preferred_element_type=jnp.float32)
    v = v + b_ref[...] + p0_ref[...] + p1_ref[...]
    o_ref[...] = jnp.maximum(v, 0.0) if act else v


def _tc_combine(x, root_w, bias, part, act):
    return pl.pallas_call(
        functools.partial(_combine_body, act=act),
        grid=(NB,),
        in_specs=[
            pl.BlockSpec((BN, D), lambda b: (b, 0)),
            pl.BlockSpec((D, D), lambda b: (0, 0)),
            pl.BlockSpec((1, D), lambda b: (0, 0)),
            pl.BlockSpec((BN, D), lambda b: (b, 0)),
            pl.BlockSpec((BN, D), lambda b: (b, 0)),
        ],
        out_specs=pl.BlockSpec((BN, D), lambda b: (b, 0)),
        out_shape=jax.ShapeDtypeStruct((N, D), jnp.float32),
    )(x, root_w, bias.reshape(1, D), part[0], part[1])


def kernel(edge_index, edge_type, node_emb, rel_w1, root_w1, bias1,
           rel_w2, root_w2, bias2):
    src = edge_index[0]
    dst = edge_index[1]
    rel = edge_type

    cnt_part = _sc_count(dst, rel)
    inv = _tc_prep(cnt_part)

    y1 = _tc_transform(node_emb, rel_w1)
    p1 = _sc_agg(y1, inv, src, dst, rel)
    x2 = _tc_combine(node_emb, root_w1, bias1, p1, act=True)

    y2 = _tc_transform(x2, rel_w2)
    p2 = _sc_agg(y2, inv, src, dst, rel)
    out = _tc_combine(x2, root_w2, bias2, p2, act=False)
    return out


# speedup vs baseline: 24.3355x; 1.4398x over previous
import functools

import jax
import jax.numpy as jnp
from jax import lax
from jax.experimental import pallas as pl
from jax.experimental.pallas import tpu as pltpu
from jax.experimental.pallas import tpu_sc as plsc

N = 10000
R = 8
D = 128
E = 320000
RN = R * N

NC = 2
NS = 16
L = 16

EDGES_PER_CORE = E // NC
EDGES_PER_TILE = EDGES_PER_CORE // NS
W = 80
NWIN = EDGES_PER_TILE // W

_MESH = plsc.VectorSubcoreMesh(core_axis_name="c", subcore_axis_name="s")
_SC_PARAMS = pltpu.CompilerParams(use_tc_tiling_on_sc=False)


CNT_ROWS_PER_TILE = RN // NS
CNT_ZROWS = 1000


@functools.partial(
    pl.kernel,
    out_type=jax.ShapeDtypeStruct((NC, RN, L), jnp.float32),
    mesh=_MESH,
    scratch_types=[
        pltpu.VMEM((W,), jnp.int32),
        pltpu.VMEM((W,), jnp.int32),
        pltpu.VMEM((W,), jnp.int32),
        pltpu.SemaphoreType.DMA,
        pltpu.SemaphoreType.DMA,
        pltpu.VMEM((W,), jnp.int32),
        pltpu.VMEM((W,), jnp.int32),
        pltpu.VMEM((W,), jnp.int32),
        pltpu.SemaphoreType.DMA,
        pltpu.SemaphoreType.DMA,
        pltpu.VMEM((W, L), jnp.float32),
        pltpu.VMEM((CNT_ZROWS, L), jnp.float32),
        pltpu.VMEM_SHARED((RN, L), jnp.float32),
    ],
    compiler_params=_SC_PARAMS,
)
def _sc_count(dst_hbm, rel_hbm, out_hbm,
              dst_a, rel_a, idx_a, sa1, sa2,
              dst_b, rel_b, idx_b, sb1, sb2,
              ones_v, zbuf, acc_sh):
    buf_a = (dst_a, rel_a, idx_a, sa1, sa2)
    buf_b = (dst_b, rel_b, idx_b, sb1, sb2)
    core = lax.axis_index("c")
    sid = lax.axis_index("s")

    @pl.loop(0, W)
    def _(k):
        ones_v[k, :] = jnp.ones((L,), jnp.float32)

    @pl.loop(0, CNT_ZROWS)
    def _(i):
        zbuf[i, :] = jnp.zeros((L,), jnp.float32)

    rowstart = sid * CNT_ROWS_PER_TILE
    for j in range(CNT_ROWS_PER_TILE // CNT_ZROWS):
        pltpu.sync_copy(zbuf, acc_sh.at[pl.ds(rowstart + j * CNT_ZROWS,
                                              CNT_ZROWS)])
    plsc.subcore_barrier()

    base = core * EDGES_PER_CORE + sid * EDGES_PER_TILE

    def start(w, buf):
        dst_v, rel_v, idx_v, s1, s2 = buf
        off = base + w * W
        pltpu.async_copy(dst_hbm.at[pl.ds(off, W)], dst_v, s1)
        pltpu.async_copy(rel_hbm.at[pl.ds(off, W)], rel_v, s2)

    def finish(w, buf):
        dst_v, rel_v, idx_v, s1, s2 = buf
        off = base + w * W
        pltpu.make_async_copy(dst_hbm.at[pl.ds(off, W)], dst_v, s1).wait()
        pltpu.make_async_copy(rel_hbm.at[pl.ds(off, W)], rel_v, s2).wait()

        @pl.loop(0, W, step=L)
        def _(j):
            sl = pl.ds(j, L)
            idx_v[sl] = rel_v[sl] * N + dst_v[sl]

        pltpu.sync_copy(ones_v, acc_sh.at[idx_v], add=True)

    start(0, buf_a)

    @pl.loop(0, NWIN - 1, step=2)
    def _(w):
        start(w + 1, buf_b)
        finish(w, buf_a)
        start(w + 2, buf_a)
        finish(w + 1, buf_b)

    finish(NWIN - 1, buf_a)

    plsc.subcore_barrier()
    for j in range(CNT_ROWS_PER_TILE // CNT_ZROWS):
        sl = pl.ds(rowstart + j * CNT_ZROWS, CNT_ZROWS)
        pltpu.sync_copy(acc_sh.at[sl], out_hbm.at[core, sl])


AGG_ROWS_PER_TILE = N // NS
AGG_ZROWS = 125


def _agg_buf_types():
    return [
        pltpu.VMEM((W,), jnp.int32),
        pltpu.VMEM((W,), jnp.int32),
        pltpu.VMEM((W,), jnp.int32),
        pltpu.VMEM((W,), jnp.int32),
        pltpu.VMEM((W,), jnp.int32),
        pltpu.VMEM((W,), jnp.int32),
        pltpu.VMEM((W, D), jnp.float32),
        pltpu.VMEM((W, L), jnp.float32),
        pltpu.SemaphoreType.DMA,
        pltpu.SemaphoreType.DMA,
        pltpu.SemaphoreType.DMA,
        pltpu.SemaphoreType.DMA,
        pltpu.SemaphoreType.DMA,
        pltpu.SemaphoreType.DMA,
    ]


@functools.partial(
    pl.kernel,
    out_type=jax.ShapeDtypeStruct((NC, N, D), jnp.float32),
    mesh=_MESH,
    scratch_types=_agg_buf_types() + _agg_buf_types() + [
        pltpu.VMEM((AGG_ZROWS, D), jnp.float32),
        pltpu.VMEM_SHARED((N, D), jnp.float32),
    ],
    compiler_params=_SC_PARAMS,
)
def _sc_agg(y_hbm, inv_hbm, src_hbm, dst_hbm, rel_hbm, out_hbm,
            *bufs_and_more):
    buf_a = bufs_and_more[0:14]
    buf_b = bufs_and_more[14:28]
    zbuf, acc_sh = bufs_and_more[28], bufs_and_more[29]
    core = lax.axis_index("c")
    sid = lax.axis_index("s")

    @pl.loop(0, AGG_ZROWS)
    def _(i):
        @pl.loop(0, D, step=L)
        def _(j):
            zbuf[i, pl.ds(j, L)] = jnp.zeros((L,), jnp.float32)

    rowstart = sid * AGG_ROWS_PER_TILE
    for j in range(AGG_ROWS_PER_TILE // AGG_ZROWS):
        pltpu.sync_copy(zbuf, acc_sh.at[pl.ds(rowstart + j * AGG_ZROWS,
                                              AGG_ZROWS)])
    plsc.subcore_barrier()

    base = core * EDGES_PER_CORE + sid * EDGES_PER_TILE

    def start_idx(w, buf):
        src_v, dst_v, rel_v = buf[0], buf[1], buf[2]
        s_src, s_dst, s_rel = buf[8], buf[9], buf[10]
        off = base + w * W
        pltpu.async_copy(src_hbm.at[pl.ds(off, W)], src_v, s_src)
        pltpu.async_copy(dst_hbm.at[pl.ds(off, W)], dst_v, s_dst)
        pltpu.async_copy(rel_hbm.at[pl.ds(off, W)], rel_v, s_rel)

    def start_gather(w, buf, pending_scatter):
        (src_v, dst_v, rel_v, gidx_v, widx_v, sdst_v, rows_v, w_v,
         s_src, s_dst, s_rel, s_rows, s_w, s_sc) = buf
        off = base + w * W
        pltpu.make_async_copy(src_hbm.at[pl.ds(off, W)], src_v, s_src).wait()
        pltpu.make_async_copy(dst_hbm.at[pl.ds(off, W)], dst_v, s_dst).wait()
        pltpu.make_async_copy(rel_hbm.at[pl.ds(off, W)], rel_v, s_rel).wait()

        @pl.loop(0, W, step=L)
        def _(j):
            sl = pl.ds(j, L)
            rel16 = rel_v[sl]
            gidx_v[sl] = rel16 * N + src_v[sl]
            widx_v[sl] = rel16 * N + dst_v[sl]

        if pending_scatter:
            pltpu.make_async_copy(rows_v, acc_sh.at[sdst_v], s_sc).wait()
        pltpu.async_copy(y_hbm.at[gidx_v], rows_v, s_rows)
        pltpu.async_copy(inv_hbm.at[widx_v], w_v, s_w)

    def finish(w, buf):
        (src_v, dst_v, rel_v, gidx_v, widx_v, sdst_v, rows_v, w_v,
         s_src, s_dst, s_rel, s_rows, s_w, s_sc) = buf
        pltpu.make_async_copy(y_hbm.at[gidx_v], rows_v, s_rows).wait()
        pltpu.make_async_copy(inv_hbm.at[widx_v], w_v, s_w).wait()

        @pl.loop(0, W, step=L)
        def _(j):
            sl = pl.ds(j, L)
            sdst_v[sl] = dst_v[sl]

        @pl.loop(0, W, unroll=4)
        def _(k):
            wk = w_v[k, :]
            for j in range(D // L):
                sl = pl.ds(j * L, L)
                rows_v[k, sl] = rows_v[k, sl] * wk

        pltpu.async_copy(rows_v, acc_sh.at[sdst_v], s_sc, add=True)

    start_idx(0, buf_a)
    start_idx(1, buf_b)
    start_gather(0, buf_a, pending_scatter=False)
    start_gather(1, buf_b, pending_scatter=False)

    @pl.loop(0, NWIN - 3, step=2)
    def _(w):
        finish(w, buf_a)
        start_idx(w + 2, buf_a)
        finish(w + 1, buf_b)
        start_idx(w + 3, buf_b)
        start_gather(w + 2, buf_a, pending_scatter=True)
        start_gather(w + 3, buf_b, pending_scatter=True)

    finish(NWIN - 3, buf_a)
    start_idx(NWIN - 1, buf_a)
    finish(NWIN - 2, buf_b)
    start_gather(NWIN - 1, buf_a, pending_scatter=True)
    finish(NWIN - 1, buf_a)

    pltpu.make_async_copy(buf_a[6], acc_sh.at[buf_a[5]], buf_a[13]).wait()
    pltpu.make_async_copy(buf_b[6], acc_sh.at[buf_b[5]], buf_b[13]).wait()

    plsc.subcore_barrier()
    for j in range(AGG_ROWS_PER_TILE // AGG_ZROWS):
        sl = pl.ds(rowstart + j * AGG_ZROWS, AGG_ZROWS)
        pltpu.sync_copy(acc_sh.at[sl], out_hbm.at[core, sl])


NB = 5
BN = N // NB


def _transform_body(x_ref, w_ref, y_ref):
    y_ref[...] = jnp.dot(x_ref[...], w_ref[0],
                         preferred_element_type=jnp.float32)


def _tc_transform(x, rel_w):
    return pl.pallas_call(
        _transform_body,
        grid=(R, NB),
        in_specs=[
            pl.BlockSpec((BN, D), lambda r, b: (b, 0)),
            pl.BlockSpec((1, D, D), lambda r, b: (r, 0, 0)),
        ],
        out_specs=pl.BlockSpec((BN, D), lambda r, b: (r * NB + b, 0)),
        out_shape=jax.ShapeDtypeStruct((RN, D), jnp.float32),
    )(x, rel_w)


def _prep_body(c0_ref, c1_ref, o_ref):
    o_ref[...] = 1.0 / jnp.maximum(c0_ref[...] + c1_ref[...], 1.0)


def _tc_prep(cnt_part):
    c = cnt_part.reshape(NC, RN * L // D, D)
    inv = pl.pallas_call(
        _prep_body,
        grid=(5,),
        in_specs=[
            pl.BlockSpec((RN * L // D // 5, D), lambda b: (b, 0)),
            pl.BlockSpec((RN * L // D // 5, D), lambda b: (b, 0)),
        ],
        out_specs=pl.BlockSpec((RN * L // D // 5, D), lambda b: (b, 0)),
        out_shape=jax.ShapeDtypeStruct((RN * L // D, D), jnp.float32),
    )(c[0], c[1])
    return inv.reshape(RN, L)


def _combine_body(x_ref, rw_ref, b_ref, p0_ref, p1_ref, o_ref, *, act):
    v = jnp.dot(x_ref[...], rw_ref[...], preferred_element_type=jnp.float32)
    v = v + b_ref[...] + p0_ref[...] + p1_ref[...]
    o_ref[...] = jnp.maximum(v, 0.0) if act else v


def _tc_combine(x, root_w, bias, part, act):
    return pl.pallas_call(
        functools.partial(_combine_body, act=act),
        grid=(NB,),
        in_specs=[
            pl.BlockSpec((BN, D), lambda b: (b, 0)),
            pl.BlockSpec((D, D), lambda b: (0, 0)),
            pl.BlockSpec((1, D), lambda b: (0, 0)),
            pl.BlockSpec((BN, D), lambda b: (b, 0)),
            pl.BlockSpec((BN, D), lambda b: (b, 0)),
        ],
        out_specs=pl.BlockSpec((BN, D), lambda b: (b, 0)),
        out_shape=jax.ShapeDtypeStruct((N, D), jnp.float32),
    )(x, root_w, bias.reshape(1, D), part[0], part[1])


def kernel(edge_index, edge_type, node_emb, rel_w1, root_w1, bias1,
           rel_w2, root_w2, bias2):
    src = edge_index[0]
    dst = edge_index[1]
    rel = edge_type

    cnt_part = _sc_count(dst, rel)
    inv = _tc_prep(cnt_part)

    y1 = _tc_transform(node_emb, rel_w1)
    p1 = _sc_agg(y1, inv, src, dst, rel)
    x2 = _tc_combine(node_emb, root_w1, bias1, p1, act=True)

    y2 = _tc_transform(x2, rel_w2)
    p2 = _sc_agg(y2, inv, src, dst, rel)
    out = _tc_combine(x2, root_w2, bias2, p2, act=False)
    return out


# speedup vs baseline: 28.8174x; 1.1842x over previous
import functools

import jax
import jax.numpy as jnp
from jax import lax
from jax.experimental import pallas as pl
from jax.experimental.pallas import tpu as pltpu
from jax.experimental.pallas import tpu_sc as plsc

N = 10000
R = 8
D = 128
E = 320000
RN = R * N

NC = 2
NS = 16
L = 16

EDGES_PER_CORE = E // NC
EDGES_PER_TILE = EDGES_PER_CORE // NS
W = 80
NWIN = EDGES_PER_TILE // W

_MESH = plsc.VectorSubcoreMesh(core_axis_name="c", subcore_axis_name="s")
_SC_PARAMS = pltpu.CompilerParams(use_tc_tiling_on_sc=False)


CNT_ROWS_PER_TILE = RN // NS
CNT_ZROWS = 1000


@functools.partial(
    pl.kernel,
    out_type=jax.ShapeDtypeStruct((NC, RN), jnp.float32),
    mesh=_MESH,
    scratch_types=[
        pltpu.VMEM((W,), jnp.int32),
        pltpu.VMEM((W,), jnp.int32),
        pltpu.VMEM((W,), jnp.int32),
        pltpu.SemaphoreType.DMA,
        pltpu.SemaphoreType.DMA,
        pltpu.VMEM((W,), jnp.int32),
        pltpu.VMEM((W,), jnp.int32),
        pltpu.VMEM((W,), jnp.int32),
        pltpu.SemaphoreType.DMA,
        pltpu.SemaphoreType.DMA,
        pltpu.VMEM((W,), jnp.float32),
        pltpu.VMEM((CNT_ROWS_PER_TILE + 8,), jnp.float32),
        pltpu.VMEM_SHARED((RN,), jnp.float32),
    ],
    compiler_params=_SC_PARAMS,
)
def _sc_count(dst_hbm, rel_hbm, out_hbm,
              dst_a, rel_a, idx_a, sa1, sa2,
              dst_b, rel_b, idx_b, sb1, sb2,
              ones_v, zbuf, acc_sh):
    buf_a = (dst_a, rel_a, idx_a, sa1, sa2)
    buf_b = (dst_b, rel_b, idx_b, sb1, sb2)
    core = lax.axis_index("c")
    sid = lax.axis_index("s")

    @pl.loop(0, W, step=L)
    def _(k):
        ones_v[pl.ds(k, L)] = jnp.ones((L,), jnp.float32)

    @pl.loop(0, CNT_ROWS_PER_TILE + 8, step=L)
    def _(i):
        zbuf[pl.ds(i, L)] = jnp.zeros((L,), jnp.float32)

    rowstart = sid * CNT_ROWS_PER_TILE
    pltpu.sync_copy(zbuf.at[pl.ds(0, CNT_ROWS_PER_TILE)],
                    acc_sh.at[pl.ds(rowstart, CNT_ROWS_PER_TILE)])
    plsc.subcore_barrier()

    base = core * EDGES_PER_CORE + sid * EDGES_PER_TILE

    def start(w, buf):
        dst_v, rel_v, idx_v, s1, s2 = buf
        off = base + w * W
        pltpu.async_copy(dst_hbm.at[pl.ds(off, W)], dst_v, s1)
        pltpu.async_copy(rel_hbm.at[pl.ds(off, W)], rel_v, s2)

    def finish(w, buf):
        dst_v, rel_v, idx_v, s1, s2 = buf
        off = base + w * W
        pltpu.make_async_copy(dst_hbm.at[pl.ds(off, W)], dst_v, s1).wait()
        pltpu.make_async_copy(rel_hbm.at[pl.ds(off, W)], rel_v, s2).wait()

        @pl.loop(0, W, step=L)
        def _(j):
            sl = pl.ds(j, L)
            idx_v[sl] = rel_v[sl] * N + dst_v[sl]

        pltpu.sync_copy(ones_v, acc_sh.at[idx_v], add=True)

    start(0, buf_a)

    @pl.loop(0, NWIN - 1, step=2)
    def _(w):
        start(w + 1, buf_b)
        finish(w, buf_a)
        start(w + 2, buf_a)
        finish(w + 1, buf_b)

    finish(NWIN - 1, buf_a)

    plsc.subcore_barrier()
    sl = pl.ds(rowstart, CNT_ROWS_PER_TILE)
    pltpu.sync_copy(acc_sh.at[sl], out_hbm.at[core, sl])


AGG_ROWS_PER_TILE = N // NS
AGG_ZROWS = 125


def _agg_buf_types():
    return [
        pltpu.VMEM((W,), jnp.int32),
        pltpu.VMEM((W,), jnp.int32),
        pltpu.VMEM((W,), jnp.int32),
        pltpu.VMEM((W,), jnp.int32),
        pltpu.VMEM((W,), jnp.int32),
        pltpu.VMEM((W,), jnp.int32),
        pltpu.VMEM((W, D), jnp.float32),
        pltpu.VMEM((W, L), jnp.float32),
        pltpu.SemaphoreType.DMA,
        pltpu.SemaphoreType.DMA,
        pltpu.SemaphoreType.DMA,
        pltpu.SemaphoreType.DMA,
        pltpu.SemaphoreType.DMA,
        pltpu.SemaphoreType.DMA,
    ]


@functools.partial(
    pl.kernel,
    out_type=jax.ShapeDtypeStruct((NC, N, D), jnp.float32),
    mesh=_MESH,
    scratch_types=_agg_buf_types() + _agg_buf_types() + [
        pltpu.VMEM((AGG_ZROWS, D), jnp.float32),
        pltpu.VMEM_SHARED((N, D), jnp.float32),
    ],
    compiler_params=_SC_PARAMS,
)
def _sc_agg(y_hbm, inv_hbm, src_hbm, dst_hbm, rel_hbm, out_hbm,
            *bufs_and_more):
    buf_a = bufs_and_more[0:14]
    buf_b = bufs_and_more[14:28]
    zbuf, acc_sh = bufs_and_more[28], bufs_and_more[29]
    core = lax.axis_index("c")
    sid = lax.axis_index("s")

    @pl.loop(0, AGG_ZROWS)
    def _(i):
        @pl.loop(0, D, step=L)
        def _(j):
            zbuf[i, pl.ds(j, L)] = jnp.zeros((L,), jnp.float32)

    rowstart = sid * AGG_ROWS_PER_TILE
    for j in range(AGG_ROWS_PER_TILE // AGG_ZROWS):
        pltpu.sync_copy(zbuf, acc_sh.at[pl.ds(rowstart + j * AGG_ZROWS,
                                              AGG_ZROWS)])
    plsc.subcore_barrier()

    base = core * EDGES_PER_CORE + sid * EDGES_PER_TILE

    def start_idx(w, buf):
        src_v, dst_v, rel_v = buf[0], buf[1], buf[2]
        s_src, s_dst, s_rel = buf[8], buf[9], buf[10]
        off = base + w * W
        pltpu.async_copy(src_hbm.at[pl.ds(off, W)], src_v, s_src)
        pltpu.async_copy(dst_hbm.at[pl.ds(off, W)], dst_v, s_dst)
        pltpu.async_copy(rel_hbm.at[pl.ds(off, W)], rel_v, s_rel)

    def start_gather(w, buf, pending_scatter):
        (src_v, dst_v, rel_v, gidx_v, widx_v, sdst_v, rows_v, w_v,
         s_src, s_dst, s_rel, s_rows, s_w, s_sc) = buf
        off = base + w * W
        pltpu.make_async_copy(src_hbm.at[pl.ds(off, W)], src_v, s_src).wait()
        pltpu.make_async_copy(dst_hbm.at[pl.ds(off, W)], dst_v, s_dst).wait()
        pltpu.make_async_copy(rel_hbm.at[pl.ds(off, W)], rel_v, s_rel).wait()

        @pl.loop(0, W, step=L)
        def _(j):
            sl = pl.ds(j, L)
            rel16 = rel_v[sl]
            gidx_v[sl] = rel16 * N + src_v[sl]
            widx_v[sl] = rel16 * N + dst_v[sl]

        if pending_scatter:
            pltpu.make_async_copy(rows_v, acc_sh.at[sdst_v], s_sc).wait()
        pltpu.async_copy(y_hbm.at[gidx_v], rows_v, s_rows)
        pltpu.async_copy(inv_hbm.at[widx_v], w_v, s_w)

    def finish(w, buf):
        (src_v, dst_v, rel_v, gidx_v, widx_v, sdst_v, rows_v, w_v,
         s_src, s_dst, s_rel, s_rows, s_w, s_sc) = buf
        pltpu.make_async_copy(y_hbm.at[gidx_v], rows_v, s_rows).wait()
        pltpu.make_async_copy(inv_hbm.at[widx_v], w_v, s_w).wait()

        @pl.loop(0, W, step=L)
        def _(j):
            sl = pl.ds(j, L)
            sdst_v[sl] = dst_v[sl]

        @pl.loop(0, W, unroll=4)
        def _(k):
            wk = w_v[k, :]
            for j in range(D // L):
                sl = pl.ds(j * L, L)
                rows_v[k, sl] = rows_v[k, sl] * wk

        pltpu.async_copy(rows_v, acc_sh.at[sdst_v], s_sc, add=True)

    start_idx(0, buf_a)
    start_idx(1, buf_b)
    start_gather(0, buf_a, pending_scatter=False)
    start_gather(1, buf_b, pending_scatter=False)

    @pl.loop(0, NWIN - 3, step=2)
    def _(w):
        finish(w, buf_a)
        start_idx(w + 2, buf_a)
        finish(w + 1, buf_b)
        start_idx(w + 3, buf_b)
        start_gather(w + 2, buf_a, pending_scatter=True)
        start_gather(w + 3, buf_b, pending_scatter=True)

    finish(NWIN - 3, buf_a)
    start_idx(NWIN - 1, buf_a)
    finish(NWIN - 2, buf_b)
    start_gather(NWIN - 1, buf_a, pending_scatter=True)
    finish(NWIN - 1, buf_a)

    pltpu.make_async_copy(buf_a[6], acc_sh.at[buf_a[5]], buf_a[13]).wait()
    pltpu.make_async_copy(buf_b[6], acc_sh.at[buf_b[5]], buf_b[13]).wait()

    plsc.subcore_barrier()
    for j in range(AGG_ROWS_PER_TILE // AGG_ZROWS):
        sl = pl.ds(rowstart + j * AGG_ZROWS, AGG_ZROWS)
        pltpu.sync_copy(acc_sh.at[sl], out_hbm.at[core, sl])


NB = 5
BN = N // NB


def _transform_body(x_ref, w_ref, y_ref):
    for r in range(R):
        y_ref[r] = jnp.dot(x_ref[...], w_ref[r],
                           preferred_element_type=jnp.float32)


def _tc_transform(x, rel_w):
    y = pl.pallas_call(
        _transform_body,
        grid=(NB,),
        in_specs=[
            pl.BlockSpec((BN, D), lambda b: (b, 0)),
            pl.BlockSpec((R, D, D), lambda b: (0, 0, 0)),
        ],
        out_specs=pl.BlockSpec((R, BN, D), lambda b: (0, b, 0)),
        out_shape=jax.ShapeDtypeStruct((R, N, D), jnp.float32),
    )(x, rel_w)
    return y.reshape(RN, D)


def _prep_body(c0_ref, c1_ref, o_ref):
    o_ref[...] = 1.0 / jnp.maximum(c0_ref[...] + c1_ref[...], 1.0)


def _tc_prep(cnt_part):
    c = cnt_part.reshape(NC, RN // D, D)
    inv = pl.pallas_call(
        _prep_body,
        grid=(1,),
        in_specs=[
            pl.BlockSpec((RN // D, D), lambda b: (0, 0)),
            pl.BlockSpec((RN // D, D), lambda b: (0, 0)),
        ],
        out_specs=pl.BlockSpec((RN // D, D), lambda b: (0, 0)),
        out_shape=jax.ShapeDtypeStruct((RN // D, D), jnp.float32),
    )(c[0], c[1])
    return jnp.broadcast_to(inv.reshape(RN, 1), (RN, L))


def _combine_body(x_ref, rw_ref, b_ref, p0_ref, p1_ref, o_ref):
    v = jnp.dot(x_ref[...], rw_ref[...], preferred_element_type=jnp.float32)
    v = v + b_ref[...] + p0_ref[...] + p1_ref[...]
    o_ref[...] = v


def _tc_combine(x, root_w, bias, part):
    return pl.pallas_call(
        _combine_body,
        grid=(NB,),
        in_specs=[
            pl.BlockSpec((BN, D), lambda b: (b, 0)),
            pl.BlockSpec((D, D), lambda b: (0, 0)),
            pl.BlockSpec((1, D), lambda b: (0, 0)),
            pl.BlockSpec((BN, D), lambda b: (b, 0)),
            pl.BlockSpec((BN, D), lambda b: (b, 0)),
        ],
        out_specs=pl.BlockSpec((BN, D), lambda b: (b, 0)),
        out_shape=jax.ShapeDtypeStruct((N, D), jnp.float32),
    )(x, root_w, bias.reshape(1, D), part[0], part[1])


def _combine_transform_body(x_ref, rw_ref, b_ref, p0_ref, p1_ref, w2_ref,
                            x2_ref, y2_ref):
    v = jnp.dot(x_ref[...], rw_ref[...], preferred_element_type=jnp.float32)
    v = v + b_ref[...] + p0_ref[...] + p1_ref[...]
    v = jnp.maximum(v, 0.0)
    x2_ref[...] = v
    for r in range(R):
        y2_ref[r] = jnp.dot(v, w2_ref[r], preferred_element_type=jnp.float32)


def _tc_combine_transform(x, root_w, bias, part, rel_w2):
    x2, y2 = pl.pallas_call(
        _combine_transform_body,
        grid=(NB,),
        in_specs=[
            pl.BlockSpec((BN, D), lambda b: (b, 0)),
            pl.BlockSpec((D, D), lambda b: (0, 0)),
            pl.BlockSpec((1, D), lambda b: (0, 0)),
            pl.BlockSpec((BN, D), lambda b: (b, 0)),
            pl.BlockSpec((BN, D), lambda b: (b, 0)),
            pl.BlockSpec((R, D, D), lambda b: (0, 0, 0)),
        ],
        out_specs=[
            pl.BlockSpec((BN, D), lambda b: (b, 0)),
            pl.BlockSpec((R, BN, D), lambda b: (0, b, 0)),
        ],
        out_shape=[
            jax.ShapeDtypeStruct((N, D), jnp.float32),
            jax.ShapeDtypeStruct((R, N, D), jnp.float32),
        ],
    )(x, root_w, bias.reshape(1, D), part[0], part[1], rel_w2)
    return x2, y2.reshape(RN, D)


def kernel(edge_index, edge_type, node_emb, rel_w1, root_w1, bias1,
           rel_w2, root_w2, bias2):
    src = edge_index[0]
    dst = edge_index[1]
    rel = edge_type

    cnt_part = _sc_count(dst, rel)
    inv = _tc_prep(cnt_part)

    y1 = _tc_transform(node_emb, rel_w1)
    p1 = _sc_agg(y1, inv, src, dst, rel)
    x2, y2 = _tc_combine_transform(node_emb, root_w1, bias1, p1, rel_w2)

    p2 = _sc_agg(y2, inv, src, dst, rel)
    out = _tc_combine(x2, root_w2, bias2, p2)
    return out
